# baseline (speedup 1.0000x reference)
import functools

import jax
import jax.numpy as jnp
from jax import lax
from jax.experimental import pallas as pl
from jax.experimental.pallas import tpu as pltpu
from jax.experimental.pallas import tpu_sc as plsc

_L = 16
_NC = 2
_NS = 16
_CH = 128
_SB = 32


def _rsqrt_vec(deg):
    half = deg * jnp.float32(0.5)
    i = plsc.bitcast(deg, jnp.int32)
    i = jnp.int32(0x5F3759DF) - (i >> 1)
    y = plsc.bitcast(i, jnp.float32)
    for _ in range(3):
        y = y * (jnp.float32(1.5) - half * y * y)
    return y


def _make_dinv_kernel(Np, rpt, F):
    slots = _CH // _L
    cols = Np // _NS
    half = cols // _NC
    mesh = plsc.VectorSubcoreMesh(core_axis_name="c", subcore_axis_name="s")

    @functools.partial(
        pl.kernel,
        out_type=(jax.ShapeDtypeStruct((Np,), jnp.float32),
                  jax.ShapeDtypeStruct((Np, F), jnp.float32)),
        mesh=mesh,
        compiler_params=pltpu.CompilerParams(needs_layout_passes=False),
        scratch_types=[
            pltpu.VMEM((rpt, _CH), jnp.int32),
            pltpu.VMEM((Np,), jnp.float32),
            pltpu.VMEM_SHARED((_NS, Np), jnp.float32),
            pltpu.VMEM((_NS, cols), jnp.float32),
            pltpu.VMEM((cols,), jnp.float32),
            pltpu.VMEM((half, F), jnp.float32),
        ],
    )
    def dinv_kernel(dst_hbm, x_hbm, dinv_hbm, xs_hbm,
                    dstbuf, countbuf, shared, redbuf, pbuf, xbuf):
        c = lax.axis_index("c")
        s = lax.axis_index("s")
        zeros = jnp.zeros((_L,), jnp.float32)

        def zbody(i, carry):
            countbuf[pl.ds(i * _L, _L)] = zeros
            return carry

        lax.fori_loop(0, Np // _L, zbody, 0)
        pltpu.sync_copy(dst_hbm.at[s], dstbuf)
        ones = jnp.ones((_L,), jnp.float32)

        def cbody(r, carry):
            for k in range(slots):
                idx = dstbuf[r, pl.ds(k * _L, _L)]
                plsc.addupdate_scatter(countbuf, [idx], ones)
            return carry

        lax.fori_loop(0, rpt, cbody, 0)
        pltpu.sync_copy(countbuf, shared.at[s])
        plsc.subcore_barrier()
        base = s * cols
        pltpu.sync_copy(shared.at[:, pl.ds(base, cols)], redbuf)

        def rbody(k, carry):
            o = k * _L
            acc = redbuf[0, pl.ds(o, _L)]
            for r in range(1, _NS):
                acc = acc + redbuf[r, pl.ds(o, _L)]
            pbuf[pl.ds(o, _L)] = _rsqrt_vec(acc + jnp.float32(1.0))
            return carry

        lax.fori_loop(0, cols // _L, rbody, 0)

        @pl.when(c == 0)
        def _():
            pltpu.sync_copy(pbuf, dinv_hbm.at[pl.ds(base, cols)])

        row0 = base + c * half
        pltpu.sync_copy(x_hbm.at[pl.ds(row0, half)], xbuf)

        def xbody(r, carry):
            v = plsc.load_gather(pbuf, [jnp.full((_L,), c * half + r,
                                                 jnp.int32)])
            for k in range(F // _L):
                xbuf[r, pl.ds(k * _L, _L)] = xbuf[r, pl.ds(k * _L, _L)] * v
            return carry

        lax.fori_loop(0, half, xbody, 0)
        pltpu.sync_copy(xbuf, xs_hbm.at[pl.ds(row0, half)])

    return dinv_kernel


def _make_agg_kernel(Np, rpt, sb):
    rpo = Np // _NS
    nsb = rpt // sb
    mesh = plsc.VectorSubcoreMesh(core_axis_name="c", subcore_axis_name="s")

    @functools.partial(
        pl.kernel,
        out_type=jax.ShapeDtypeStruct((_NC, Np, 128), jnp.float32),
        mesh=mesh,
        compiler_params=pltpu.CompilerParams(needs_layout_passes=False),
        scratch_types=[
            pltpu.VMEM((sb, _CH), jnp.int32),
            pltpu.VMEM((sb, _CH), jnp.int32),
            pltpu.VMEM((2, _CH, 128), jnp.float32),
            pltpu.VMEM_SHARED((Np, 128), jnp.float32),
            pltpu.SemaphoreType.DMA((2,)),
            pltpu.SemaphoreType.DMA((2,)),
        ],
    )
    def agg_kernel(hp_hbm, src_hbm, dst_hbm, out_hbm,
                   srcbuf, dstbuf, rows, acc, gsem, ssem):
        c = lax.axis_index("c")
        s = lax.axis_index("s")
        zeros = jnp.zeros((_L,), jnp.float32)

        def zbody(r, carry):
            for k in range(128 // _L):
                rows[0, r, pl.ds(k * _L, _L)] = zeros
            return carry

        lax.fori_loop(0, _CH, zbody, 0)
        for t in range(rpo // _CH):
            pltpu.async_copy(rows.at[0], acc.at[pl.ds(s * rpo + t * _CH, _CH)],
                             gsem.at[0])
        for t in range(rpo // _CH):
            pltpu.make_async_copy(rows.at[0],
                                  acc.at[pl.ds(s * rpo + t * _CH, _CH)],
                                  gsem.at[0]).wait()
        plsc.subcore_barrier()

        def _wait_gather(j):
            m = j % 2
            pltpu.make_async_copy(hp_hbm.at[srcbuf.at[j]], rows.at[m],
                                  gsem.at[m]).wait()

        def _wait_scatter(j):
            m = j % 2
            pltpu.make_async_copy(rows.at[m], acc.at[dstbuf.at[j]],
                                  ssem.at[m]).wait()

        def sbody(b, carry):
            pltpu.sync_copy(src_hbm.at[c, s, pl.ds(b * sb, sb)], srcbuf)
            pltpu.sync_copy(dst_hbm.at[c, s, pl.ds(b * sb, sb)], dstbuf)
            for j in range(sb):
                m = j % 2
                if j >= 2:
                    _wait_scatter(j - 2)
                pltpu.async_copy(hp_hbm.at[srcbuf.at[j]], rows.at[m],
                                 gsem.at[m])
                if j >= 1:
                    _wait_gather(j - 1)
                    pltpu.async_copy(rows.at[(j - 1) % 2],
                                     acc.at[dstbuf.at[j - 1]],
                                     ssem.at[(j - 1) % 2], add=True)
            _wait_scatter(sb - 2)
            _wait_gather(sb - 1)
            pltpu.async_copy(rows.at[(sb - 1) % 2], acc.at[dstbuf.at[sb - 1]],
                             ssem.at[(sb - 1) % 2], add=True)
            _wait_scatter(sb - 1)
            return carry

        lax.fori_loop(0, nsb, sbody, 0)
        plsc.subcore_barrier()
        pltpu.sync_copy(acc.at[pl.ds(s * rpo, rpo)],
                        out_hbm.at[c, pl.ds(s * rpo, rpo)])

    return agg_kernel


def _split2(h):
    return jnp.stack([h[:, :128], h[:, 128:]], axis=0)


def _mmmid_body(a_ref, xs_ref, dinv_ref, b_ref, w1_ref, w2_ref, out_ref):
    dv = dinv_ref[...]
    u = a_ref[0] + a_ref[1] + xs_ref[...]
    g = jnp.dot(u, w1_ref[...], preferred_element_type=jnp.float32)
    z = jnp.maximum(g * dv + b_ref[...], 0.0)
    m = jnp.dot(z, w2_ref[...], preferred_element_type=jnp.float32) * dv
    out_ref[...] = _split2(m)


def _fin_body(agg_ref, hp_ref, dinv_ref, b_ref, wl_ref, bl_ref, out_ref):
    dv = dinv_ref[...]
    z = jnp.concatenate([agg_ref[0] + hp_ref[0], agg_ref[1] + hp_ref[1]], axis=1)
    z = jnp.maximum(z * dv + b_ref[...], 0.0)
    out_ref[...] = (
        jnp.dot(z, wl_ref[...], preferred_element_type=jnp.float32) + bl_ref[...]
    )


def kernel(x, edge_index, W1, b1, W2, b2, Wl, bl):
    N, F = x.shape
    H = W1.shape[1]
    C = Wl.shape[1]
    E = edge_index.shape[1]
    R = 256
    Np = -(-N // 1280) * 1280

    Ep = -(-E // (_NS * _SB * _CH)) * (_NS * _SB * _CH)
    npad = Ep - E
    pad = N + jnp.arange(npad, dtype=jnp.int32) % (Np - N)
    src_flat = jnp.concatenate([edge_index[0].astype(jnp.int32), pad])
    dst_flat = jnp.concatenate([edge_index[1].astype(jnp.int32), pad])
    rpt = Ep // (_NS * _CH)
    rpt1 = rpt // _NC
    src_a1 = src_flat.reshape(_NC, _NS, rpt1, _CH)
    dst_a1 = dst_flat.reshape(_NC, _NS, rpt1, _CH)
    src_r = src_flat.reshape(_NS, rpt, _CH)
    dst_r = dst_flat.reshape(_NS, rpt, _CH)
    src_a2 = jnp.stack([src_r, src_r + Np])
    dst_a2 = jnp.stack([dst_r, dst_r])

    x_pad = jnp.pad(x, ((0, Np - N), (0, 0)))
    wl_pad = jnp.pad(Wl, ((0, 0), (0, 128 - C)))
    bl_pad = jnp.pad(bl, (0, 128 - C)).reshape(1, 128)
    b1_2d = b1.reshape(1, H)
    b2_2d = b2.reshape(1, H)

    dinv, xs = _make_dinv_kernel(Np, rpt, F)(dst_r, x_pad)
    dinv_col = dinv.reshape(Np, 1)

    G = Np // R
    agg1 = _make_agg_kernel(Np, rpt1, 40)(xs, src_a1, dst_a1)

    hp2 = pl.pallas_call(
        _mmmid_body,
        grid=(G,),
        in_specs=[
            pl.BlockSpec((_NC, R, 128), lambda i: (0, i, 0)),
            pl.BlockSpec((R, F), lambda i: (i, 0)),
            pl.BlockSpec((R, 1), lambda i: (i, 0)),
            pl.BlockSpec((1, H), lambda i: (0, 0)),
            pl.BlockSpec((F, H), lambda i: (0, 0)),
            pl.BlockSpec((H, H), lambda i: (0, 0)),
        ],
        out_specs=pl.BlockSpec((_NC, R, 128), lambda i: (0, i, 0)),
        out_shape=jax.ShapeDtypeStruct((_NC, Np, 128), jnp.float32),
    )(agg1, xs, dinv_col, b1_2d, W1, W2)

    agg2 = _make_agg_kernel(Np, rpt, 40)(hp2.reshape(_NC * Np, 128),
                                         src_a2, dst_a2)

    logits = pl.pallas_call(
        _fin_body,
        grid=(G,),
        in_specs=[
            pl.BlockSpec((_NC, R, 128), lambda i: (0, i, 0)),
            pl.BlockSpec((_NC, R, 128), lambda i: (0, i, 0)),
            pl.BlockSpec((R, 1), lambda i: (i, 0)),
            pl.BlockSpec((1, H), lambda i: (0, 0)),
            pl.BlockSpec((H, 128), lambda i: (0, 0)),
            pl.BlockSpec((1, 128), lambda i: (0, 0)),
        ],
        out_specs=pl.BlockSpec((R, 128), lambda i: (i, 0)),
        out_shape=jax.ShapeDtypeStruct((Np, 128), jnp.float32),
    )(agg2, hp2, dinv_col, b2_2d, wl_pad, bl_pad)

    return logits[:N, :C]


# speedup vs baseline: 1.0908x; 1.0908x over previous
import functools

import jax
import jax.numpy as jnp
from jax import lax
from jax.experimental import pallas as pl
from jax.experimental.pallas import tpu as pltpu
from jax.experimental.pallas import tpu_sc as plsc

_L = 16
_NC = 2
_NS = 16
_CH = 128
_SB = 32


def _rsqrt_vec(deg):
    half = deg * jnp.float32(0.5)
    i = plsc.bitcast(deg, jnp.int32)
    i = jnp.int32(0x5F3759DF) - (i >> 1)
    y = plsc.bitcast(i, jnp.float32)
    for _ in range(3):
        y = y * (jnp.float32(1.5) - half * y * y)
    return y


def _make_dinv_kernel(Np, rpt, F):
    slots = _CH // _L
    cols = Np // _NS
    half = cols // _NC
    mesh = plsc.VectorSubcoreMesh(core_axis_name="c", subcore_axis_name="s")

    @functools.partial(
        pl.kernel,
        out_type=(jax.ShapeDtypeStruct((Np,), jnp.float32),
                  jax.ShapeDtypeStruct((Np, F), jnp.float32)),
        mesh=mesh,
        compiler_params=pltpu.CompilerParams(needs_layout_passes=False),
        scratch_types=[
            pltpu.VMEM((rpt, _CH), jnp.int32),
            pltpu.VMEM((Np,), jnp.float32),
            pltpu.VMEM_SHARED((_NS, Np), jnp.float32),
            pltpu.VMEM((_NS, cols), jnp.float32),
            pltpu.VMEM((cols,), jnp.float32),
            pltpu.VMEM((half, F), jnp.float32),
        ],
    )
    def dinv_kernel(dst_hbm, x_hbm, dinv_hbm, xs_hbm,
                    dstbuf, countbuf, shared, redbuf, pbuf, xbuf):
        c = lax.axis_index("c")
        s = lax.axis_index("s")
        zeros = jnp.zeros((_L,), jnp.float32)

        def zbody(i, carry):
            countbuf[pl.ds(i * _L, _L)] = zeros
            return carry

        lax.fori_loop(0, Np // _L, zbody, 0)
        pltpu.sync_copy(dst_hbm.at[s], dstbuf)
        ones = jnp.ones((_L,), jnp.float32)

        def cbody(r, carry):
            for k in range(slots):
                idx = dstbuf[r, pl.ds(k * _L, _L)]
                plsc.addupdate_scatter(countbuf, [idx], ones)
            return carry

        lax.fori_loop(0, rpt, cbody, 0)
        pltpu.sync_copy(countbuf, shared.at[s])
        plsc.subcore_barrier()
        base = s * cols
        pltpu.sync_copy(shared.at[:, pl.ds(base, cols)], redbuf)

        def rbody(k, carry):
            o = k * _L
            acc = redbuf[0, pl.ds(o, _L)]
            for r in range(1, _NS):
                acc = acc + redbuf[r, pl.ds(o, _L)]
            pbuf[pl.ds(o, _L)] = _rsqrt_vec(acc + jnp.float32(1.0))
            return carry

        lax.fori_loop(0, cols // _L, rbody, 0)

        @pl.when(c == 0)
        def _():
            pltpu.sync_copy(pbuf, dinv_hbm.at[pl.ds(base, cols)])

        row0 = base + c * half
        pltpu.sync_copy(x_hbm.at[pl.ds(row0, half)], xbuf)

        def xbody(r, carry):
            v = plsc.load_gather(pbuf, [jnp.full((_L,), c * half + r,
                                                 jnp.int32)])
            for k in range(F // _L):
                xbuf[r, pl.ds(k * _L, _L)] = xbuf[r, pl.ds(k * _L, _L)] * v
            return carry

        lax.fori_loop(0, half, xbody, 0)
        pltpu.sync_copy(xbuf, xs_hbm.at[pl.ds(row0, half)])

    return dinv_kernel


def _make_agg_kernel(Np, rpt, sb, edge_split):
    rpo = Np // _NS
    nsb = rpt // sb
    mesh = plsc.VectorSubcoreMesh(core_axis_name="c", subcore_axis_name="s")

    @functools.partial(
        pl.kernel,
        out_type=jax.ShapeDtypeStruct((_NC, Np, 128), jnp.float32),
        mesh=mesh,
        compiler_params=pltpu.CompilerParams(needs_layout_passes=False),
        scratch_types=[
            pltpu.VMEM((sb, _CH), jnp.int32),
            pltpu.VMEM((sb, _CH), jnp.int32),
            pltpu.VMEM((2, _CH, 128), jnp.float32),
            pltpu.VMEM_SHARED((Np, 128), jnp.float32),
            pltpu.SemaphoreType.DMA((2,)),
            pltpu.SemaphoreType.DMA((2,)),
        ],
    )
    def agg_kernel(hp_hbm, src_hbm, dst_hbm, out_hbm,
                   srcbuf, dstbuf, rows, acc, gsem, ssem):
        c = lax.axis_index("c")
        s = lax.axis_index("s")
        zeros = jnp.zeros((_L,), jnp.float32)

        def zbody(r, carry):
            for k in range(128 // _L):
                rows[0, r, pl.ds(k * _L, _L)] = zeros
            return carry

        lax.fori_loop(0, _CH, zbody, 0)
        for t in range(rpo // _CH):
            pltpu.async_copy(rows.at[0], acc.at[pl.ds(s * rpo + t * _CH, _CH)],
                             gsem.at[0])
        for t in range(rpo // _CH):
            pltpu.make_async_copy(rows.at[0],
                                  acc.at[pl.ds(s * rpo + t * _CH, _CH)],
                                  gsem.at[0]).wait()
        plsc.subcore_barrier()

        def _wait_gather(j):
            m = j % 2
            pltpu.make_async_copy(hp_hbm.at[srcbuf.at[j]], rows.at[m],
                                  gsem.at[m]).wait()

        def _wait_scatter(j):
            m = j % 2
            pltpu.make_async_copy(rows.at[m], acc.at[dstbuf.at[j]],
                                  ssem.at[m]).wait()

        def sbody(b, carry):
            pltpu.sync_copy(src_hbm.at[c, s, pl.ds(b * sb, sb)], srcbuf)
            if edge_split:
                pltpu.sync_copy(dst_hbm.at[c, s, pl.ds(b * sb, sb)], dstbuf)
            else:
                pltpu.sync_copy(dst_hbm.at[s, pl.ds(b * sb, sb)], dstbuf)
            for j in range(sb):
                m = j % 2
                if j >= 2:
                    _wait_scatter(j - 2)
                pltpu.async_copy(hp_hbm.at[srcbuf.at[j]], rows.at[m],
                                 gsem.at[m])
                if j >= 1:
                    _wait_gather(j - 1)
                    pltpu.async_copy(rows.at[(j - 1) % 2],
                                     acc.at[dstbuf.at[j - 1]],
                                     ssem.at[(j - 1) % 2], add=True)
            _wait_scatter(sb - 2)
            _wait_gather(sb - 1)
            pltpu.async_copy(rows.at[(sb - 1) % 2], acc.at[dstbuf.at[sb - 1]],
                             ssem.at[(sb - 1) % 2], add=True)
            _wait_scatter(sb - 1)
            return carry

        lax.fori_loop(0, nsb, sbody, 0)
        plsc.subcore_barrier()
        pltpu.sync_copy(acc.at[pl.ds(s * rpo, rpo)],
                        out_hbm.at[c, pl.ds(s * rpo, rpo)])

    return agg_kernel


def _split2(h):
    return jnp.stack([h[:, :128], h[:, 128:]], axis=0)


def _mmmid_body(a_ref, xs_ref, dinv_ref, b_ref, w1_ref, w2_ref, out_ref):
    dv = dinv_ref[...]
    u = a_ref[0] + a_ref[1] + xs_ref[...]
    g = jnp.dot(u, w1_ref[...], preferred_element_type=jnp.float32)
    z = jnp.maximum(g * dv + b_ref[...], 0.0)
    m = jnp.dot(z, w2_ref[...], preferred_element_type=jnp.float32) * dv
    out_ref[...] = _split2(m)


def _fin_body(agg_ref, hp_ref, dinv_ref, b_ref, wl_ref, bl_ref, out_ref):
    dv = dinv_ref[...]
    z = jnp.concatenate([agg_ref[0] + hp_ref[0], agg_ref[1] + hp_ref[1]], axis=1)
    z = jnp.maximum(z * dv + b_ref[...], 0.0)
    out_ref[...] = (
        jnp.dot(z, wl_ref[...], preferred_element_type=jnp.float32) + bl_ref[...]
    )


def kernel(x, edge_index, W1, b1, W2, b2, Wl, bl):
    N, F = x.shape
    H = W1.shape[1]
    C = Wl.shape[1]
    E = edge_index.shape[1]
    R = 1024
    Np = -(-N // 1280) * 1280

    Ep = -(-E // (_NS * _SB * _CH)) * (_NS * _SB * _CH)
    npad = Ep - E
    pad = N + jnp.arange(npad, dtype=jnp.int32) % (Np - N)
    src_flat = jnp.concatenate([edge_index[0].astype(jnp.int32), pad])
    dst_flat = jnp.concatenate([edge_index[1].astype(jnp.int32), pad])
    rpt = Ep // (_NS * _CH)
    rpt1 = rpt // _NC
    src_a1 = src_flat.reshape(_NC, _NS, rpt1, _CH)
    dst_a1 = dst_flat.reshape(_NC, _NS, rpt1, _CH)
    src_r = src_flat.reshape(_NS, rpt, _CH)
    dst_r = dst_flat.reshape(_NS, rpt, _CH)
    src_a2 = jnp.stack([src_r, src_r + Np])

    x_pad = jnp.pad(x, ((0, Np - N), (0, 0)))
    b1_2d = b1.reshape(1, H)
    b2_2d = b2.reshape(1, H)

    dinv, xs = _make_dinv_kernel(Np, rpt, F)(dst_r, x_pad)
    dinv_col = dinv.reshape(Np, 1)

    G = Np // R
    agg1 = _make_agg_kernel(Np, rpt1, 40, True)(xs, src_a1, dst_a1)

    hp2 = pl.pallas_call(
        _mmmid_body,
        grid=(G,),
        in_specs=[
            pl.BlockSpec((_NC, R, 128), lambda i: (0, i, 0)),
            pl.BlockSpec((R, F), lambda i: (i, 0)),
            pl.BlockSpec((R, 1), lambda i: (i, 0)),
            pl.BlockSpec((1, H), lambda i: (0, 0)),
            pl.BlockSpec((F, H), lambda i: (0, 0)),
            pl.BlockSpec((H, H), lambda i: (0, 0)),
        ],
        out_specs=pl.BlockSpec((_NC, R, 128), lambda i: (0, i, 0)),
        out_shape=jax.ShapeDtypeStruct((_NC, Np, 128), jnp.float32),
    )(agg1, xs, dinv_col, b1_2d, W1, W2)

    agg2 = _make_agg_kernel(Np, rpt, 40, False)(hp2.reshape(_NC * Np, 128),
                                                src_a2, dst_r)

    Rf = 1000
    logits = pl.pallas_call(
        _fin_body,
        grid=(N // Rf,),
        in_specs=[
            pl.BlockSpec((_NC, Rf, 128), lambda i: (0, i, 0)),
            pl.BlockSpec((_NC, Rf, 128), lambda i: (0, i, 0)),
            pl.BlockSpec((Rf, 1), lambda i: (i, 0)),
            pl.BlockSpec((1, H), lambda i: (0, 0)),
            pl.BlockSpec((H, C), lambda i: (0, 0)),
            pl.BlockSpec((1, C), lambda i: (0, 0)),
        ],
        out_specs=pl.BlockSpec((Rf, C), lambda i: (i, 0)),
        out_shape=jax.ShapeDtypeStruct((N, C), jnp.float32),
    )(agg2, hp2, dinv_col, b2_2d, Wl, bl.reshape(1, C))

    return logits
